# Initial kernel scaffold; baseline (speedup 1.0000x reference)
import jax
import jax.numpy as jnp
from jax.experimental import pallas as pl


def kernel(y, edge_attr, edge_index, W_pre, b_pre, W_upd, b_upd):
    raise NotImplementedError("write your pallas kernel here")


# speedup vs baseline: 1.4128x; 1.4128x over previous
import functools

import jax
import jax.numpy as jnp
from jax import lax
from jax.experimental import pallas as pl
from jax.experimental.pallas import tpu as pltpu
from jax.experimental.pallas import tpu_sc as plsc

N = 10000
E = 320000
D_NODE = 128
D_EDGE = 16
D_OUT = 128
D_MSG = 144
DP = 160

NC = 2
NS = 16
NW = NC * NS
EPW = E // NW
CHUNK = 40
NCHUNK = EPW // CHUNK
NPAD = 10240
ZROWS = NPAD // NS


def _ewb_body(e_ref, w_ref, b_ref, o_ref):
    o_ref[...] = (
        jnp.dot(e_ref[...], w_ref[...], preferred_element_type=jnp.float32)
        + b_ref[...]
    )


def _yw_body(y_ref, w_ref, o_ref):
    o_ref[...] = jnp.dot(y_ref[...], w_ref[...],
                         preferred_element_type=jnp.float32)


def _final_body(z_ref, w_ref, b_ref, o_ref):
    zs = z_ref[0] + z_ref[1]
    cnt = jnp.maximum(zs[:, D_MSG:D_MSG + 1], 1.0)
    zm = zs[:, :D_MSG] / cnt
    o_ref[...] = jnp.maximum(
        jnp.dot(zm, w_ref[...], preferred_element_type=jnp.float32)
        + b_ref[...],
        0.0,
    )


def _sc_body(yw_hbm, ewb_hbm, src_hbm, dst_hbm, out_hbm,
             zsh, srcv, dstv, ewbv, ywv, sem):
    c = lax.axis_index("c")
    s = lax.axis_index("s")
    wid = c * NS + s

    def zero_row(r, carry):
        for j in range(DP // 16):
            ewbv[r, pl.ds(j * 16, 16)] = jnp.zeros((16,), jnp.float32)
        return carry
    lax.fori_loop(0, CHUNK, zero_row, 0)
    for t in range(ZROWS // CHUNK):
        pltpu.sync_copy(ewbv, zsh.at[pl.ds(s * ZROWS + t * CHUNK, CHUNK)])
    plsc.subcore_barrier()

    base = wid * EPW

    def chunk_body(i, carry):
        off = base + i * CHUNK
        pltpu.sync_copy(src_hbm.at[pl.ds(off, CHUNK)], srcv)
        pltpu.sync_copy(dst_hbm.at[pl.ds(off, CHUNK)], dstv)
        pltpu.sync_copy(ewb_hbm.at[pl.ds(off, CHUNK)], ewbv)
        pltpu.async_copy(yw_hbm.at[srcv], ywv, sem).wait()

        def row(r, rc):
            for j in range(D_MSG // 16):
                a = ewbv[r, pl.ds(j * 16, 16)]
                b = ywv[r, pl.ds(j * 16, 16)]
                ewbv[r, pl.ds(j * 16, 16)] = jnp.maximum(a + b, 0.0)
            return rc
        lax.fori_loop(0, CHUNK, row, 0)

        pltpu.sync_copy(ewbv, zsh.at[dstv], add=True)
        return carry

    lax.fori_loop(0, NCHUNK, chunk_body, 0)
    plsc.subcore_barrier()

    for t in range(ZROWS // CHUNK):
        r0 = s * ZROWS + t * CHUNK
        pltpu.sync_copy(zsh.at[pl.ds(r0, CHUNK)], ewbv)
        pltpu.sync_copy(ewbv, out_hbm.at[c, pl.ds(r0, CHUNK)])


_sc_aggregate = pl.kernel(
    _sc_body,
    out_type=jax.ShapeDtypeStruct((NC, NPAD, DP), jnp.float32),
    mesh=plsc.VectorSubcoreMesh(core_axis_name="c", subcore_axis_name="s"),
    compiler_params=pltpu.CompilerParams(use_tc_tiling_on_sc=False),
    scratch_types=[
        pltpu.VMEM_SHARED((NPAD, DP), jnp.float32),
        pltpu.VMEM((CHUNK,), jnp.int32),
        pltpu.VMEM((CHUNK,), jnp.int32),
        pltpu.VMEM((CHUNK, DP), jnp.float32),
        pltpu.VMEM((CHUNK, D_MSG), jnp.float32),
        pltpu.SemaphoreType.DMA,
    ],
)


def kernel(y, edge_attr, edge_index, W_pre, b_pre, W_upd, b_upd):
    src = edge_index[0]
    dst = edge_index[1]
    W1 = W_pre[:D_NODE]
    W2p = jnp.zeros((D_EDGE, DP), jnp.float32).at[:, :D_MSG].set(
        W_pre[D_NODE:])
    bp = jnp.zeros((1, DP), jnp.float32).at[0, :D_MSG].set(b_pre)
    bp = bp.at[0, D_MSG].set(1.0)

    BE = 3200
    ewb = pl.pallas_call(
        _ewb_body,
        grid=(E // BE,),
        in_specs=[
            pl.BlockSpec((BE, D_EDGE), lambda i: (i, 0)),
            pl.BlockSpec((D_EDGE, DP), lambda i: (0, 0)),
            pl.BlockSpec((1, DP), lambda i: (0, 0)),
        ],
        out_specs=pl.BlockSpec((BE, DP), lambda i: (i, 0)),
        out_shape=jax.ShapeDtypeStruct((E, DP), jnp.float32),
    )(edge_attr, W2p, bp)

    BN = 2000
    yw = pl.pallas_call(
        _yw_body,
        grid=(N // BN,),
        in_specs=[
            pl.BlockSpec((BN, D_NODE), lambda i: (i, 0)),
            pl.BlockSpec((D_NODE, D_MSG), lambda i: (0, 0)),
        ],
        out_specs=pl.BlockSpec((BN, D_MSG), lambda i: (i, 0)),
        out_shape=jax.ShapeDtypeStruct((N, D_MSG), jnp.float32),
    )(y, W1)

    zacc = _sc_aggregate(yw, ewb, src, dst)

    BZ = 1000
    h = pl.pallas_call(
        _final_body,
        grid=(N // BZ,),
        in_specs=[
            pl.BlockSpec((NC, BZ, DP), lambda i: (0, i, 0)),
            pl.BlockSpec((D_MSG, D_OUT), lambda i: (0, 0)),
            pl.BlockSpec((1, D_OUT), lambda i: (0, 0)),
        ],
        out_specs=pl.BlockSpec((BZ, D_OUT), lambda i: (i, 0)),
        out_shape=jax.ShapeDtypeStruct((N, D_OUT), jnp.float32),
    )(zacc, W_upd, b_upd.reshape(1, D_OUT))

    return h


# speedup vs baseline: 1.9748x; 1.3978x over previous
import jax
import jax.numpy as jnp
from jax import lax
from jax.experimental import pallas as pl
from jax.experimental.pallas import tpu as pltpu
from jax.experimental.pallas import tpu_sc as plsc

N = 10000
E = 320000
D_NODE = 128
D_EDGE = 16
D_OUT = 128
D_MSG = 144
DP = 160

NC = 2
NS = 16
NW = NC * NS
EPW = E // NW
CHUNK = 40
NCHUNK = EPW // CHUNK
NPAD = 10240
ZROWS = NPAD // NS


def _ewb_body(e_ref, w_ref, b_ref, o_ref):
    o_ref[...] = (
        lax.dot_general(e_ref[...], w_ref[...], (((0,), (0,)), ((), ())),
                        preferred_element_type=jnp.float32)
        + b_ref[...]
    )


def _yw_body(y_ref, w_ref, o_ref):
    o_ref[...] = jnp.dot(y_ref[...], w_ref[...],
                         preferred_element_type=jnp.float32)


def _final_body(z_ref, w_ref, b_ref, o_ref):
    zs = z_ref[0] + z_ref[1]
    cnt = jnp.maximum(zs[:, D_MSG:D_MSG + 1], 1.0)
    zm = zs[:, :D_MSG] / cnt
    o_ref[...] = jnp.maximum(
        jnp.dot(zm, w_ref[...], preferred_element_type=jnp.float32)
        + b_ref[...],
        0.0,
    )


def _sc_body(yw_hbm, ewb_hbm, src_hbm, dst_hbm, out_hbm,
             zsh, srcv0, srcv1, dstv0, dstv1, ewbv0, ewbv1, ywv0, ywv1,
             sem_in0, sem_in1, sem_g0, sem_g1):
    c = lax.axis_index("c")
    s = lax.axis_index("s")
    wid = c * NS + s
    base = wid * EPW

    srcv = (srcv0, srcv1)
    dstv = (dstv0, dstv1)
    ewbv = (ewbv0, ewbv1)
    ywv = (ywv0, ywv1)
    sem_in = (sem_in0, sem_in1)
    sem_g = (sem_g0, sem_g1)

    def zero_row(r, carry):
        for j in range(DP // 16):
            ewbv0[r, pl.ds(j * 16, 16)] = jnp.zeros((16,), jnp.float32)
        return carry
    lax.fori_loop(0, CHUNK, zero_row, 0)
    for t in range(ZROWS // CHUNK):
        pltpu.sync_copy(ewbv0, zsh.at[pl.ds(s * ZROWS + t * CHUNK, CHUNK)])
    plsc.subcore_barrier()

    def in_copies(g, b):
        off = base + g * CHUNK
        return (
            pltpu.make_async_copy(src_hbm.at[pl.ds(off, CHUNK)], srcv[b],
                                  sem_in[b]),
            pltpu.make_async_copy(dst_hbm.at[pl.ds(off, CHUNK)], dstv[b],
                                  sem_in[b]),
            pltpu.make_async_copy(ewb_hbm.at[pl.ds(off, CHUNK)], ewbv[b],
                                  sem_in[b]),
        )

    def in_start(g, b):
        for cp in in_copies(g, b):
            cp.start()

    def in_wait(g, b):
        for cp in in_copies(g, b):
            cp.wait()

    def gather(b):
        return pltpu.make_async_copy(yw_hbm.at[srcv[b]], ywv[b], sem_g[b])

    def compute(b):
        def row(r, rc):
            for j in range(D_MSG // 16):
                a = ewbv[b][r, pl.ds(j * 16, 16)]
                v = ywv[b][r, pl.ds(j * 16, 16)]
                ewbv[b][r, pl.ds(j * 16, 16)] = jnp.maximum(a + v, 0.0)
            return rc
        lax.fori_loop(0, CHUNK, row, 0)

    in_start(0, 0)
    in_start(1, 1)
    in_wait(0, 0)
    gather(0).start()

    def step(g, b):
        nb = 1 - b

        @pl.when(g + 1 < NCHUNK)
        def _():
            in_wait(g + 1, nb)
            gather(nb).start()

        gather(b).wait()
        compute(b)
        pltpu.sync_copy(ewbv[b], zsh.at[dstv[b]], add=True)

        @pl.when(g + 2 < NCHUNK)
        def _():
            in_start(g + 2, b)

    def pair_body(i, carry):
        step(2 * i, 0)
        step(2 * i + 1, 1)
        return carry

    lax.fori_loop(0, NCHUNK // 2, pair_body, 0)
    plsc.subcore_barrier()

    for t in range(ZROWS // CHUNK):
        r0 = s * ZROWS + t * CHUNK
        pltpu.sync_copy(zsh.at[pl.ds(r0, CHUNK)], ewbv0)
        pltpu.sync_copy(ewbv0, out_hbm.at[c, pl.ds(r0, CHUNK)])


_sc_aggregate = pl.kernel(
    _sc_body,
    out_type=jax.ShapeDtypeStruct((NC, NPAD, DP), jnp.float32),
    mesh=plsc.VectorSubcoreMesh(core_axis_name="c", subcore_axis_name="s"),
    compiler_params=pltpu.CompilerParams(use_tc_tiling_on_sc=False),
    scratch_types=[
        pltpu.VMEM_SHARED((NPAD, DP), jnp.float32),
        pltpu.VMEM((CHUNK,), jnp.int32),
        pltpu.VMEM((CHUNK,), jnp.int32),
        pltpu.VMEM((CHUNK,), jnp.int32),
        pltpu.VMEM((CHUNK,), jnp.int32),
        pltpu.VMEM((CHUNK, DP), jnp.float32),
        pltpu.VMEM((CHUNK, DP), jnp.float32),
        pltpu.VMEM((CHUNK, D_MSG), jnp.float32),
        pltpu.VMEM((CHUNK, D_MSG), jnp.float32),
        pltpu.SemaphoreType.DMA,
        pltpu.SemaphoreType.DMA,
        pltpu.SemaphoreType.DMA,
        pltpu.SemaphoreType.DMA,
    ],
)


def kernel(y, edge_attr, edge_index, W_pre, b_pre, W_upd, b_upd):
    src = edge_index[0]
    dst = edge_index[1]
    eT = edge_attr.T
    W1 = W_pre[:D_NODE]
    W2p = jnp.zeros((D_EDGE, DP), jnp.float32).at[:, :D_MSG].set(
        W_pre[D_NODE:])
    bp = jnp.zeros((1, DP), jnp.float32).at[0, :D_MSG].set(b_pre)
    bp = bp.at[0, D_MSG].set(1.0)

    BE = 3200
    ewb = pl.pallas_call(
        _ewb_body,
        grid=(E // BE,),
        in_specs=[
            pl.BlockSpec((D_EDGE, BE), lambda i: (0, i)),
            pl.BlockSpec((D_EDGE, DP), lambda i: (0, 0)),
            pl.BlockSpec((1, DP), lambda i: (0, 0)),
        ],
        out_specs=pl.BlockSpec((BE, DP), lambda i: (i, 0)),
        out_shape=jax.ShapeDtypeStruct((E, DP), jnp.float32),
    )(eT, W2p, bp)

    BN = 2000
    yw = pl.pallas_call(
        _yw_body,
        grid=(N // BN,),
        in_specs=[
            pl.BlockSpec((BN, D_NODE), lambda i: (i, 0)),
            pl.BlockSpec((D_NODE, D_MSG), lambda i: (0, 0)),
        ],
        out_specs=pl.BlockSpec((BN, D_MSG), lambda i: (i, 0)),
        out_shape=jax.ShapeDtypeStruct((N, D_MSG), jnp.float32),
    )(y, W1)

    zacc = _sc_aggregate(yw, ewb, src, dst)

    BZ = 1000
    h = pl.pallas_call(
        _final_body,
        grid=(N // BZ,),
        in_specs=[
            pl.BlockSpec((NC, BZ, DP), lambda i: (0, i, 0)),
            pl.BlockSpec((D_MSG, D_OUT), lambda i: (0, 0)),
            pl.BlockSpec((1, D_OUT), lambda i: (0, 0)),
        ],
        out_specs=pl.BlockSpec((BZ, D_OUT), lambda i: (i, 0)),
        out_shape=jax.ShapeDtypeStruct((N, D_OUT), jnp.float32),
    )(zacc, W_upd, b_upd.reshape(1, D_OUT))

    return h


# speedup vs baseline: 2.4834x; 1.2576x over previous
import jax
import jax.numpy as jnp
from jax import lax
from jax.experimental import pallas as pl
from jax.experimental.pallas import tpu as pltpu
from jax.experimental.pallas import tpu_sc as plsc

N = 10000
E = 320000
D_NODE = 128
D_EDGE = 16
D_OUT = 128
D_MSG = 144

NC = 2
NS = 16
NW = NC * NS
EPW = E // NW
CHUNK = 40
NCHUNK = EPW // CHUNK
ZROWS = N // NS
ZTAIL = ZROWS % CHUNK
TTILES = E // 128


def _edge_body(eT_ref, wa_ref, ba_ref, v_ref, oa_ref, ot_ref):
    eT = eT_ref[...]
    oa_ref[...] = (
        lax.dot_general(eT, wa_ref[...], (((0,), (0,)), ((), ())),
                        preferred_element_type=jnp.float32)
        + ba_ref[...]
    )
    ft = lax.dot_general(v_ref[...], eT, (((0,), (0,)), ((), ())),
                         preferred_element_type=jnp.float32)
    for ct in range(ot_ref.shape[0]):
        ot_ref[ct] = ft[:, 128 * ct:128 * (ct + 1)]


def _yw_body(y_ref, w_ref, b_ref, o_ref):
    o_ref[...] = jnp.dot(y_ref[...], w_ref[...],
                         preferred_element_type=jnp.float32) + b_ref[...]


def _final_body(z_ref, c_ref, w_ref, b_ref, o_ref):
    zs = z_ref[0] + z_ref[1]
    cnt = jnp.maximum(jnp.sum(c_ref[0], axis=(0, 1)), 1.0)
    zm = zs / cnt[:, None]
    o_ref[...] = jnp.maximum(
        jnp.dot(zm, w_ref[...], preferred_element_type=jnp.float32)
        + b_ref[...],
        0.0,
    )


def _sc_body(yw_hbm, ewba_hbm, tail_hbm, src_hbm, dst_hbm, z_hbm, cnt_hbm,
             zsh, srcv0, srcv1, dstv0, dstv1, eav0, eav1, tlv0, tlv1,
             mbuf0, mbuf1, hist, sem_in0, sem_in1, sem_g0, sem_g1):
    c = lax.axis_index("c")
    s = lax.axis_index("s")
    wid = c * NS + s
    base = wid * EPW

    srcv = (srcv0, srcv1)
    dstv = (dstv0, dstv1)
    eav = (eav0, eav1)
    tlv = (tlv0, tlv1)
    mbuf = (mbuf0, mbuf1)
    sem_in = (sem_in0, sem_in1)
    sem_g = (sem_g0, sem_g1)

    zero16 = jnp.zeros((16,), jnp.float32)
    one16 = jnp.full((16,), 1.0, jnp.float32)
    iota16 = lax.iota(jnp.int32, 16)

    def zero_row(r, carry):
        for j in range(D_MSG // 16):
            mbuf0[r, pl.ds(j * 16, 16)] = zero16
        return carry
    lax.fori_loop(0, CHUNK, zero_row, 0)

    def zero_hist(i, carry):
        hist[pl.ds(i * 16, 16)] = zero16
        return carry
    lax.fori_loop(0, N // 16, zero_hist, 0)

    for t in range(ZROWS // CHUNK):
        pltpu.sync_copy(mbuf0, zsh.at[pl.ds(s * ZROWS + t * CHUNK, CHUNK)])
    pltpu.sync_copy(mbuf0.at[pl.ds(0, ZTAIL)],
                    zsh.at[pl.ds(s * ZROWS + (ZROWS // CHUNK) * CHUNK,
                                 ZTAIL)])
    plsc.subcore_barrier()

    def chunk_off(g):
        off = base + g * CHUNK
        ct0 = jnp.minimum(lax.shift_right_logical(off, 7), TTILES - 2)
        return off, ct0

    def in_copies(g, b):
        off, ct0 = chunk_off(g)
        return (
            pltpu.make_async_copy(src_hbm.at[pl.ds(off, CHUNK)], srcv[b],
                                  sem_in[b]),
            pltpu.make_async_copy(dst_hbm.at[pl.ds(off, CHUNK)], dstv[b],
                                  sem_in[b]),
            pltpu.make_async_copy(ewba_hbm.at[pl.ds(off, CHUNK)], eav[b],
                                  sem_in[b]),
            pltpu.make_async_copy(tail_hbm.at[pl.ds(ct0, 2)], tlv[b],
                                  sem_in[b]),
        )

    def in_start(g, b):
        for cp in in_copies(g, b):
            cp.start()

    def in_wait(g, b):
        for cp in in_copies(g, b):
            cp.wait()

    def gather(b):
        return pltpu.make_async_copy(yw_hbm.at[srcv[b]], mbuf[b], sem_g[b])

    def compute(g, b):
        off, ct0 = chunk_off(g)
        c_off = off - lax.shift_left(ct0, 7)

        def row(r, rc):
            for j in range(8):
                a = eav[b][r, pl.ds(j * 16, 16)]
                v = mbuf[b][r, pl.ds(j * 16, 16)]
                mbuf[b][r, pl.ds(j * 16, 16)] = jnp.maximum(a + v, 0.0)
            q = c_off + r
            t = lax.shift_right_logical(q, 7)
            l = q - lax.shift_left(t, 7)
            tail = plsc.load_gather(
                tlv[b],
                [jnp.full((16,), t, jnp.int32), iota16,
                 jnp.full((16,), l, jnp.int32)])
            v8 = mbuf[b][r, pl.ds(128, 16)]
            mbuf[b][r, pl.ds(128, 16)] = jnp.maximum(tail + v8, 0.0)
            return rc
        lax.fori_loop(0, CHUNK, row, 0)

        plsc.addupdate_scatter(hist, [dstv[b][pl.ds(0, 16)]], one16)
        plsc.addupdate_scatter(hist, [dstv[b][pl.ds(16, 16)]], one16)
        plsc.addupdate_scatter(hist, [dstv[b][pl.ds(24, 16)]], one16,
                               mask=iota16 >= 8)

    in_start(0, 0)
    in_start(1, 1)
    in_wait(0, 0)
    gather(0).start()

    def step(g, b):
        nb = 1 - b

        @pl.when(g + 1 < NCHUNK)
        def _():
            in_wait(g + 1, nb)
            gather(nb).start()

        gather(b).wait()
        compute(g, b)
        pltpu.sync_copy(mbuf[b], zsh.at[dstv[b]], add=True)

        @pl.when(g + 2 < NCHUNK)
        def _():
            in_start(g + 2, b)

    def pair_body(i, carry):
        step(2 * i, 0)
        step(2 * i + 1, 1)
        return carry

    lax.fori_loop(0, NCHUNK // 2, pair_body, 0)
    plsc.subcore_barrier()

    for t in range(ZROWS // CHUNK):
        r0 = s * ZROWS + t * CHUNK
        pltpu.sync_copy(zsh.at[pl.ds(r0, CHUNK)], mbuf0)
        pltpu.sync_copy(mbuf0, z_hbm.at[c, pl.ds(r0, CHUNK)])
    r0 = s * ZROWS + (ZROWS // CHUNK) * CHUNK
    pltpu.sync_copy(zsh.at[pl.ds(r0, ZTAIL)], mbuf0.at[pl.ds(0, ZTAIL)])
    pltpu.sync_copy(mbuf0.at[pl.ds(0, ZTAIL)], z_hbm.at[c, pl.ds(r0, ZTAIL)])
    pltpu.sync_copy(hist, cnt_hbm.at[c, s])


_sc_aggregate = pl.kernel(
    _sc_body,
    out_type=(
        jax.ShapeDtypeStruct((NC, N, D_MSG), jnp.float32),
        jax.ShapeDtypeStruct((NC, NS, N), jnp.float32),
    ),
    mesh=plsc.VectorSubcoreMesh(core_axis_name="c", subcore_axis_name="s"),
    compiler_params=pltpu.CompilerParams(use_tc_tiling_on_sc=False,
                                         needs_layout_passes=False),
    scratch_types=[
        pltpu.VMEM_SHARED((N, D_MSG), jnp.float32),
        pltpu.VMEM((CHUNK,), jnp.int32),
        pltpu.VMEM((CHUNK,), jnp.int32),
        pltpu.VMEM((CHUNK,), jnp.int32),
        pltpu.VMEM((CHUNK,), jnp.int32),
        pltpu.VMEM((CHUNK, D_NODE), jnp.float32),
        pltpu.VMEM((CHUNK, D_NODE), jnp.float32),
        pltpu.VMEM((2, 16, 128), jnp.float32),
        pltpu.VMEM((2, 16, 128), jnp.float32),
        pltpu.VMEM((CHUNK, D_MSG), jnp.float32),
        pltpu.VMEM((CHUNK, D_MSG), jnp.float32),
        pltpu.VMEM((N,), jnp.float32),
        pltpu.SemaphoreType.DMA,
        pltpu.SemaphoreType.DMA,
        pltpu.SemaphoreType.DMA,
        pltpu.SemaphoreType.DMA,
    ],
)


def kernel(y, edge_attr, edge_index, W_pre, b_pre, W_upd, b_upd):
    src = edge_index[0]
    dst = edge_index[1]
    eT = edge_attr.T
    W1 = W_pre[:D_NODE]
    byw = jnp.zeros((1, D_MSG), jnp.float32).at[0, D_NODE:].set(
        b_pre[D_NODE:])
    W2a = W_pre[D_NODE:, :D_NODE]
    ba = b_pre[:D_NODE].reshape(1, D_NODE)
    Vmsg = W_pre[D_NODE:, D_NODE:]

    BE = 6400
    BT = BE // 128
    ewba, tailt = pl.pallas_call(
        _edge_body,
        grid=(E // BE,),
        in_specs=[
            pl.BlockSpec((D_EDGE, BE), lambda i: (0, i)),
            pl.BlockSpec((D_EDGE, D_NODE), lambda i: (0, 0)),
            pl.BlockSpec((1, D_NODE), lambda i: (0, 0)),
            pl.BlockSpec((D_EDGE, D_EDGE), lambda i: (0, 0)),
        ],
        out_specs=[
            pl.BlockSpec((BE, D_NODE), lambda i: (i, 0)),
            pl.BlockSpec((BT, 16, 128), lambda i: (i, 0, 0)),
        ],
        out_shape=[
            jax.ShapeDtypeStruct((E, D_NODE), jnp.float32),
            jax.ShapeDtypeStruct((TTILES, 16, 128), jnp.float32),
        ],
    )(eT, W2a, ba, Vmsg)

    BN = 2000
    yw = pl.pallas_call(
        _yw_body,
        grid=(N // BN,),
        in_specs=[
            pl.BlockSpec((BN, D_NODE), lambda i: (i, 0)),
            pl.BlockSpec((D_NODE, D_MSG), lambda i: (0, 0)),
            pl.BlockSpec((1, D_MSG), lambda i: (0, 0)),
        ],
        out_specs=pl.BlockSpec((BN, D_MSG), lambda i: (i, 0)),
        out_shape=jax.ShapeDtypeStruct((N, D_MSG), jnp.float32),
    )(y, W1, byw)

    zacc, cnts = _sc_aggregate(yw, ewba, tailt, src, dst)

    BZ = 1000
    h = pl.pallas_call(
        _final_body,
        grid=(N // BZ,),
        in_specs=[
            pl.BlockSpec((NC, BZ, D_MSG), lambda i: (0, i, 0)),
            pl.BlockSpec((1, NC, NS, BZ), lambda i: (i, 0, 0, 0)),
            pl.BlockSpec((D_MSG, D_OUT), lambda i: (0, 0)),
            pl.BlockSpec((1, D_OUT), lambda i: (0, 0)),
        ],
        out_specs=pl.BlockSpec((BZ, D_OUT), lambda i: (i, 0)),
        out_shape=jax.ShapeDtypeStruct((N, D_OUT), jnp.float32),
    )(zacc, cnts.reshape(NC, NS, N // BZ, BZ).transpose(2, 0, 1, 3),
      W_upd, b_upd.reshape(1, D_OUT))

    return h


# speedup vs baseline: 3.7703x; 1.5182x over previous
import jax
import jax.numpy as jnp
from jax import lax
from jax.experimental import pallas as pl
from jax.experimental.pallas import tpu as pltpu
from jax.experimental.pallas import tpu_sc as plsc

N = 10000
E = 320000
D_NODE = 128
D_EDGE = 16
D_OUT = 128
D_MSG = 144

NC = 2
NS = 16
NW = NC * NS
EPW = E // NW
CHUNK = 40
NCHUNK = EPW // CHUNK
ZROWS = N // NS
ZTAIL = ZROWS % CHUNK
TTILES = E // 128


def _edge_body(eT_ref, wa_ref, ba_ref, v_ref, oa_ref, ot_ref):
    eT = eT_ref[...]
    oa_ref[...] = (
        lax.dot_general(eT, wa_ref[...], (((0,), (0,)), ((), ())),
                        preferred_element_type=jnp.float32)
        + ba_ref[...]
    )
    ft = lax.dot_general(v_ref[...], eT, (((0,), (0,)), ((), ())),
                         preferred_element_type=jnp.float32)
    for ct in range(ot_ref.shape[0]):
        ot_ref[ct] = ft[:, 128 * ct:128 * (ct + 1)]


def _yw_body(y_ref, w_ref, b_ref, o_ref):
    o_ref[...] = jnp.dot(y_ref[...], w_ref[...],
                         preferred_element_type=jnp.float32) + b_ref[...]


def _final_body(z_ref, c_ref, w_ref, b_ref, o_ref):
    zs = z_ref[0] + z_ref[1]
    cnt = jnp.maximum(jnp.sum(c_ref[0], axis=(0, 1)), 1.0)
    zm = zs / cnt[:, None]
    o_ref[...] = jnp.maximum(
        jnp.dot(zm, w_ref[...], preferred_element_type=jnp.float32)
        + b_ref[...],
        0.0,
    )


def _sc_body(yw_hbm, ewba_hbm, tail_hbm, src_hbm, dst_hbm, z_hbm, cnt_hbm,
             zsh, srcv0, srcv1, dstv0, dstv1, eav0, eav1, tlv0, tlv1,
             mbuf0, mbuf1, hist, sem_in0, sem_in1, sem_g0, sem_g1):
    c = lax.axis_index("c")
    s = lax.axis_index("s")
    wid = c * NS + s
    base = wid * EPW

    srcv = (srcv0, srcv1)
    dstv = (dstv0, dstv1)
    eav = (eav0, eav1)
    tlv = (tlv0, tlv1)
    mbuf = (mbuf0, mbuf1)
    sem_in = (sem_in0, sem_in1)
    sem_g = (sem_g0, sem_g1)

    zero16 = jnp.zeros((16,), jnp.float32)
    one16 = jnp.full((16,), 1.0, jnp.float32)
    iota16 = lax.iota(jnp.int32, 16)

    def zero_row(r, carry):
        for j in range(D_MSG // 16):
            mbuf0[r, pl.ds(j * 16, 16)] = zero16
        return carry
    lax.fori_loop(0, CHUNK, zero_row, 0)

    def zero_hist(i, carry):
        hist[pl.ds(i * 16, 16)] = zero16
        return carry
    lax.fori_loop(0, N // 16, zero_hist, 0)

    for t in range(ZROWS // CHUNK):
        pltpu.sync_copy(mbuf0, zsh.at[pl.ds(s * ZROWS + t * CHUNK, CHUNK)])
    pltpu.sync_copy(mbuf0.at[pl.ds(0, ZTAIL)],
                    zsh.at[pl.ds(s * ZROWS + (ZROWS // CHUNK) * CHUNK,
                                 ZTAIL)])
    plsc.subcore_barrier()

    def chunk_off(g):
        off = base + g * CHUNK
        ct0 = jnp.minimum(lax.shift_right_logical(off, 7), TTILES - 2)
        return off, ct0

    def in_copies(g, b):
        off, ct0 = chunk_off(g)
        return (
            pltpu.make_async_copy(src_hbm.at[pl.ds(off, CHUNK)], srcv[b],
                                  sem_in[b]),
            pltpu.make_async_copy(dst_hbm.at[pl.ds(off, CHUNK)], dstv[b],
                                  sem_in[b]),
            pltpu.make_async_copy(ewba_hbm.at[pl.ds(off, CHUNK)], eav[b],
                                  sem_in[b]),
            pltpu.make_async_copy(tail_hbm.at[pl.ds(ct0, 2)], tlv[b],
                                  sem_in[b]),
        )

    def in_start(g, b):
        for cp in in_copies(g, b):
            cp.start()

    def in_wait(g, b):
        for cp in in_copies(g, b):
            cp.wait()

    def gather(b):
        return pltpu.make_async_copy(yw_hbm.at[srcv[b]], mbuf[b], sem_g[b])

    def compute(g, b):
        off, ct0 = chunk_off(g)
        c_off = off - lax.shift_left(ct0, 7)

        @plsc.parallel_loop(0, CHUNK, unroll=2)
        def row(r):
            q = c_off + r
            t = lax.shift_right_logical(q, 7)
            l = q - lax.shift_left(t, 7)
            tail = plsc.load_gather(
                tlv[b],
                [jnp.full((16,), t, jnp.int32), iota16,
                 jnp.full((16,), l, jnp.int32)])
            ea = [eav[b][r, pl.ds(j * 16, 16)] for j in range(8)]
            yv = [mbuf[b][r, pl.ds(j * 16, 16)] for j in range(9)]
            res = [jnp.maximum(ea[j] + yv[j], 0.0) for j in range(8)]
            res.append(jnp.maximum(tail + yv[8], 0.0))
            for j in range(9):
                mbuf[b][r, pl.ds(j * 16, 16)] = res[j]

        plsc.addupdate_scatter(hist, [dstv[b][pl.ds(0, 16)]], one16)
        plsc.addupdate_scatter(hist, [dstv[b][pl.ds(16, 16)]], one16)
        plsc.addupdate_scatter(hist, [dstv[b][pl.ds(24, 16)]], one16,
                               mask=iota16 >= 8)

    in_start(0, 0)
    in_start(1, 1)
    in_wait(0, 0)
    gather(0).start()

    def step(g, b):
        nb = 1 - b

        @pl.when(g + 1 < NCHUNK)
        def _():
            in_wait(g + 1, nb)
            gather(nb).start()

        gather(b).wait()
        compute(g, b)
        pltpu.sync_copy(mbuf[b], zsh.at[dstv[b]], add=True)

        @pl.when(g + 2 < NCHUNK)
        def _():
            in_start(g + 2, b)

    def pair_body(i, carry):
        step(2 * i, 0)
        step(2 * i + 1, 1)
        return carry

    lax.fori_loop(0, NCHUNK // 2, pair_body, 0)
    plsc.subcore_barrier()

    for t in range(ZROWS // CHUNK):
        r0 = s * ZROWS + t * CHUNK
        pltpu.sync_copy(zsh.at[pl.ds(r0, CHUNK)], mbuf0)
        pltpu.sync_copy(mbuf0, z_hbm.at[c, pl.ds(r0, CHUNK)])
    r0 = s * ZROWS + (ZROWS // CHUNK) * CHUNK
    pltpu.sync_copy(zsh.at[pl.ds(r0, ZTAIL)], mbuf0.at[pl.ds(0, ZTAIL)])
    pltpu.sync_copy(mbuf0.at[pl.ds(0, ZTAIL)], z_hbm.at[c, pl.ds(r0, ZTAIL)])
    pltpu.sync_copy(hist, cnt_hbm.at[c, s])


_sc_aggregate = pl.kernel(
    _sc_body,
    out_type=(
        jax.ShapeDtypeStruct((NC, N, D_MSG), jnp.float32),
        jax.ShapeDtypeStruct((NC, NS, N), jnp.float32),
    ),
    mesh=plsc.VectorSubcoreMesh(core_axis_name="c", subcore_axis_name="s"),
    compiler_params=pltpu.CompilerParams(use_tc_tiling_on_sc=False,
                                         needs_layout_passes=False),
    scratch_types=[
        pltpu.VMEM_SHARED((N, D_MSG), jnp.float32),
        pltpu.VMEM((CHUNK,), jnp.int32),
        pltpu.VMEM((CHUNK,), jnp.int32),
        pltpu.VMEM((CHUNK,), jnp.int32),
        pltpu.VMEM((CHUNK,), jnp.int32),
        pltpu.VMEM((CHUNK, D_NODE), jnp.float32),
        pltpu.VMEM((CHUNK, D_NODE), jnp.float32),
        pltpu.VMEM((2, 16, 128), jnp.float32),
        pltpu.VMEM((2, 16, 128), jnp.float32),
        pltpu.VMEM((CHUNK, D_MSG), jnp.float32),
        pltpu.VMEM((CHUNK, D_MSG), jnp.float32),
        pltpu.VMEM((N,), jnp.float32),
        pltpu.SemaphoreType.DMA,
        pltpu.SemaphoreType.DMA,
        pltpu.SemaphoreType.DMA,
        pltpu.SemaphoreType.DMA,
    ],
)


def kernel(y, edge_attr, edge_index, W_pre, b_pre, W_upd, b_upd):
    src = edge_index[0]
    dst = edge_index[1]
    eT = edge_attr.T
    W1 = W_pre[:D_NODE]
    byw = jnp.zeros((1, D_MSG), jnp.float32).at[0, D_NODE:].set(
        b_pre[D_NODE:])
    W2a = W_pre[D_NODE:, :D_NODE]
    ba = b_pre[:D_NODE].reshape(1, D_NODE)
    Vmsg = W_pre[D_NODE:, D_NODE:]

    BE = 6400
    BT = BE // 128
    ewba, tailt = pl.pallas_call(
        _edge_body,
        grid=(E // BE,),
        in_specs=[
            pl.BlockSpec((D_EDGE, BE), lambda i: (0, i)),
            pl.BlockSpec((D_EDGE, D_NODE), lambda i: (0, 0)),
            pl.BlockSpec((1, D_NODE), lambda i: (0, 0)),
            pl.BlockSpec((D_EDGE, D_EDGE), lambda i: (0, 0)),
        ],
        out_specs=[
            pl.BlockSpec((BE, D_NODE), lambda i: (i, 0)),
            pl.BlockSpec((BT, 16, 128), lambda i: (i, 0, 0)),
        ],
        out_shape=[
            jax.ShapeDtypeStruct((E, D_NODE), jnp.float32),
            jax.ShapeDtypeStruct((TTILES, 16, 128), jnp.float32),
        ],
    )(eT, W2a, ba, Vmsg)

    BN = 2000
    yw = pl.pallas_call(
        _yw_body,
        grid=(N // BN,),
        in_specs=[
            pl.BlockSpec((BN, D_NODE), lambda i: (i, 0)),
            pl.BlockSpec((D_NODE, D_MSG), lambda i: (0, 0)),
            pl.BlockSpec((1, D_MSG), lambda i: (0, 0)),
        ],
        out_specs=pl.BlockSpec((BN, D_MSG), lambda i: (i, 0)),
        out_shape=jax.ShapeDtypeStruct((N, D_MSG), jnp.float32),
    )(y, W1, byw)

    zacc, cnts = _sc_aggregate(yw, ewba, tailt, src, dst)

    BZ = 1000
    h = pl.pallas_call(
        _final_body,
        grid=(N // BZ,),
        in_specs=[
            pl.BlockSpec((NC, BZ, D_MSG), lambda i: (0, i, 0)),
            pl.BlockSpec((1, NC, NS, BZ), lambda i: (i, 0, 0, 0)),
            pl.BlockSpec((D_MSG, D_OUT), lambda i: (0, 0)),
            pl.BlockSpec((1, D_OUT), lambda i: (0, 0)),
        ],
        out_specs=pl.BlockSpec((BZ, D_OUT), lambda i: (i, 0)),
        out_shape=jax.ShapeDtypeStruct((N, D_OUT), jnp.float32),
    )(zacc, cnts.reshape(NC, NS, N // BZ, BZ).transpose(2, 0, 1, 3),
      W_upd, b_upd.reshape(1, D_OUT))

    return h


# speedup vs baseline: 4.1421x; 1.0986x over previous
import jax
import jax.numpy as jnp
from jax import lax
from jax.experimental import pallas as pl
from jax.experimental.pallas import tpu as pltpu
from jax.experimental.pallas import tpu_sc as plsc

N = 10000
E = 320000
D_NODE = 128
D_EDGE = 16
D_OUT = 128
D_MSG = 144

NC = 2
NS = 16
NW = NC * NS
EPW = E // NW
CHUNK = 40
NCHUNK = EPW // CHUNK
ZROWS = N // NS
ZTAIL = ZROWS % CHUNK
TTILES = E // 128


def _edge_body(eT_ref, wa_ref, ba_ref, v_ref, oa_ref, ot_ref):
    eT = eT_ref[...]
    oa_ref[...] = (
        lax.dot_general(eT, wa_ref[...], (((0,), (0,)), ((), ())),
                        preferred_element_type=jnp.float32)
        + ba_ref[...]
    )
    ft = lax.dot_general(v_ref[...], eT, (((0,), (0,)), ((), ())),
                         preferred_element_type=jnp.float32)
    for ct in range(ot_ref.shape[0]):
        ot_ref[ct] = ft[:, 128 * ct:128 * (ct + 1)]


def _yw_body(y_ref, w_ref, b_ref, o_ref):
    o_ref[...] = jnp.dot(y_ref[...], w_ref[...],
                         preferred_element_type=jnp.float32) + b_ref[...]


def _final_body(z_ref, c_ref, w_ref, b_ref, o_ref):
    zs = z_ref[0] + z_ref[1]
    cnt = jnp.maximum(jnp.sum(c_ref[0], axis=(0, 1)), 1.0)
    zm = zs / cnt[:, None]
    o_ref[...] = jnp.maximum(
        jnp.dot(zm, w_ref[...], preferred_element_type=jnp.float32)
        + b_ref[...],
        0.0,
    )


def _sc_body(yw_hbm, ewba_hbm, tail_hbm, src_hbm, dst_hbm, z_hbm, cnt_hbm,
             zsh, srcv0, srcv1, dstv0, dstv1, dsts0, dsts1, eav0, eav1,
             tlv0, tlv1, mbuf0, mbuf1, hist,
             sem_in0, sem_in1, sem_g0, sem_g1, sem_s0, sem_s1):
    c = lax.axis_index("c")
    s = lax.axis_index("s")
    wid = c * NS + s
    base = wid * EPW

    srcv = (srcv0, srcv1)
    dstv = (dstv0, dstv1)
    eav = (eav0, eav1)
    tlv = (tlv0, tlv1)
    mbuf = (mbuf0, mbuf1)
    dsts = (dsts0, dsts1)
    sem_in = (sem_in0, sem_in1)
    sem_g = (sem_g0, sem_g1)
    sem_s = (sem_s0, sem_s1)

    zero16 = jnp.zeros((16,), jnp.float32)
    one16 = jnp.full((16,), 1.0, jnp.float32)
    iota16 = lax.iota(jnp.int32, 16)

    def zero_row(r, carry):
        for j in range(D_MSG // 16):
            mbuf0[r, pl.ds(j * 16, 16)] = zero16
        return carry
    lax.fori_loop(0, CHUNK, zero_row, 0)

    def zero_hist(i, carry):
        hist[pl.ds(i * 16, 16)] = zero16
        return carry
    lax.fori_loop(0, N // 16, zero_hist, 0)

    for t in range(ZROWS // CHUNK):
        pltpu.sync_copy(mbuf0, zsh.at[pl.ds(s * ZROWS + t * CHUNK, CHUNK)])
    pltpu.sync_copy(mbuf0.at[pl.ds(0, ZTAIL)],
                    zsh.at[pl.ds(s * ZROWS + (ZROWS // CHUNK) * CHUNK,
                                 ZTAIL)])
    plsc.subcore_barrier()

    def chunk_off(g):
        off = base + g * CHUNK
        ct0 = jnp.minimum(lax.shift_right_logical(off, 7), TTILES - 2)
        return off, ct0

    def in_copies(g, b):
        off, ct0 = chunk_off(g)
        return (
            pltpu.make_async_copy(src_hbm.at[pl.ds(off, CHUNK)], srcv[b],
                                  sem_in[b]),
            pltpu.make_async_copy(dst_hbm.at[pl.ds(off, CHUNK)], dstv[b],
                                  sem_in[b]),
            pltpu.make_async_copy(ewba_hbm.at[pl.ds(off, CHUNK)], eav[b],
                                  sem_in[b]),
            pltpu.make_async_copy(tail_hbm.at[pl.ds(ct0, 2)], tlv[b],
                                  sem_in[b]),
        )

    def in_start(g, b):
        for cp in in_copies(g, b):
            cp.start()

    def in_wait(g, b):
        for cp in in_copies(g, b):
            cp.wait()

    def gather(b):
        return pltpu.make_async_copy(yw_hbm.at[srcv[b]], mbuf[b], sem_g[b])

    def compute(g, b):
        off, ct0 = chunk_off(g)
        c_off = off - lax.shift_left(ct0, 7)

        @plsc.parallel_loop(0, CHUNK, unroll=2)
        def row(r):
            q = c_off + r
            t = lax.shift_right_logical(q, 7)
            l = q - lax.shift_left(t, 7)
            tail = plsc.load_gather(
                tlv[b],
                [jnp.full((16,), t, jnp.int32), iota16,
                 jnp.full((16,), l, jnp.int32)])
            ea = [eav[b][r, pl.ds(j * 16, 16)] for j in range(8)]
            yv = [mbuf[b][r, pl.ds(j * 16, 16)] for j in range(9)]
            res = [jnp.maximum(ea[j] + yv[j], 0.0) for j in range(8)]
            res.append(jnp.maximum(tail + yv[8], 0.0))
            for j in range(9):
                mbuf[b][r, pl.ds(j * 16, 16)] = res[j]

        plsc.addupdate_scatter(hist, [dstv[b][pl.ds(0, 16)]], one16)
        plsc.addupdate_scatter(hist, [dstv[b][pl.ds(16, 16)]], one16)
        plsc.addupdate_scatter(hist, [dstv[b][pl.ds(24, 16)]], one16,
                               mask=iota16 >= 8)

    in_start(0, 0)
    in_start(1, 1)
    in_wait(0, 0)
    gather(0).start()

    def scatter_wait(b):
        pltpu.make_async_copy(mbuf[b], zsh.at[dsts[b]], sem_s[b]).wait()

    def step(g, b):
        nb = 1 - b

        @pl.when(g + 1 < NCHUNK)
        def _():
            in_wait(g + 1, nb)

            @pl.when(g >= 1)
            def _():
                scatter_wait(nb)

            gather(nb).start()

        gather(b).wait()
        compute(g, b)
        dsts[b][pl.ds(0, 16)] = dstv[b][pl.ds(0, 16)]
        dsts[b][pl.ds(16, 16)] = dstv[b][pl.ds(16, 16)]
        dsts[b][pl.ds(24, 16)] = dstv[b][pl.ds(24, 16)]
        pltpu.async_copy(mbuf[b], zsh.at[dsts[b]], sem_s[b], add=True)

        @pl.when(g + 2 < NCHUNK)
        def _():
            in_start(g + 2, b)

    def pair_body(i, carry):
        step(2 * i, 0)
        step(2 * i + 1, 1)
        return carry

    lax.fori_loop(0, NCHUNK // 2, pair_body, 0)
    scatter_wait(0)
    scatter_wait(1)
    plsc.subcore_barrier()

    for t in range(ZROWS // CHUNK):
        r0 = s * ZROWS + t * CHUNK
        pltpu.sync_copy(zsh.at[pl.ds(r0, CHUNK)], mbuf0)
        pltpu.sync_copy(mbuf0, z_hbm.at[c, pl.ds(r0, CHUNK)])
    r0 = s * ZROWS + (ZROWS // CHUNK) * CHUNK
    pltpu.sync_copy(zsh.at[pl.ds(r0, ZTAIL)], mbuf0.at[pl.ds(0, ZTAIL)])
    pltpu.sync_copy(mbuf0.at[pl.ds(0, ZTAIL)], z_hbm.at[c, pl.ds(r0, ZTAIL)])
    pltpu.sync_copy(hist, cnt_hbm.at[c, s])


_sc_aggregate = pl.kernel(
    _sc_body,
    out_type=(
        jax.ShapeDtypeStruct((NC, N, D_MSG), jnp.float32),
        jax.ShapeDtypeStruct((NC, NS, N), jnp.float32),
    ),
    mesh=plsc.VectorSubcoreMesh(core_axis_name="c", subcore_axis_name="s"),
    compiler_params=pltpu.CompilerParams(use_tc_tiling_on_sc=False,
                                         needs_layout_passes=False),
    scratch_types=[
        pltpu.VMEM_SHARED((N, D_MSG), jnp.float32),
        pltpu.VMEM((CHUNK,), jnp.int32),
        pltpu.VMEM((CHUNK,), jnp.int32),
        pltpu.VMEM((CHUNK,), jnp.int32),
        pltpu.VMEM((CHUNK,), jnp.int32),
        pltpu.VMEM((CHUNK,), jnp.int32),
        pltpu.VMEM((CHUNK,), jnp.int32),
        pltpu.VMEM((CHUNK, D_NODE), jnp.float32),
        pltpu.VMEM((CHUNK, D_NODE), jnp.float32),
        pltpu.VMEM((2, 16, 128), jnp.float32),
        pltpu.VMEM((2, 16, 128), jnp.float32),
        pltpu.VMEM((CHUNK, D_MSG), jnp.float32),
        pltpu.VMEM((CHUNK, D_MSG), jnp.float32),
        pltpu.VMEM((N,), jnp.float32),
        pltpu.SemaphoreType.DMA,
        pltpu.SemaphoreType.DMA,
        pltpu.SemaphoreType.DMA,
        pltpu.SemaphoreType.DMA,
        pltpu.SemaphoreType.DMA,
        pltpu.SemaphoreType.DMA,
    ],
)


def kernel(y, edge_attr, edge_index, W_pre, b_pre, W_upd, b_upd):
    src = edge_index[0]
    dst = edge_index[1]
    eT = edge_attr.T
    W1 = W_pre[:D_NODE]
    byw = jnp.zeros((1, D_MSG), jnp.float32).at[0, D_NODE:].set(
        b_pre[D_NODE:])
    W2a = W_pre[D_NODE:, :D_NODE]
    ba = b_pre[:D_NODE].reshape(1, D_NODE)
    Vmsg = W_pre[D_NODE:, D_NODE:]

    BE = 6400
    BT = BE // 128
    ewba, tailt = pl.pallas_call(
        _edge_body,
        grid=(E // BE,),
        in_specs=[
            pl.BlockSpec((D_EDGE, BE), lambda i: (0, i)),
            pl.BlockSpec((D_EDGE, D_NODE), lambda i: (0, 0)),
            pl.BlockSpec((1, D_NODE), lambda i: (0, 0)),
            pl.BlockSpec((D_EDGE, D_EDGE), lambda i: (0, 0)),
        ],
        out_specs=[
            pl.BlockSpec((BE, D_NODE), lambda i: (i, 0)),
            pl.BlockSpec((BT, 16, 128), lambda i: (i, 0, 0)),
        ],
        out_shape=[
            jax.ShapeDtypeStruct((E, D_NODE), jnp.float32),
            jax.ShapeDtypeStruct((TTILES, 16, 128), jnp.float32),
        ],
    )(eT, W2a, ba, Vmsg)

    BN = 2000
    yw = pl.pallas_call(
        _yw_body,
        grid=(N // BN,),
        in_specs=[
            pl.BlockSpec((BN, D_NODE), lambda i: (i, 0)),
            pl.BlockSpec((D_NODE, D_MSG), lambda i: (0, 0)),
            pl.BlockSpec((1, D_MSG), lambda i: (0, 0)),
        ],
        out_specs=pl.BlockSpec((BN, D_MSG), lambda i: (i, 0)),
        out_shape=jax.ShapeDtypeStruct((N, D_MSG), jnp.float32),
    )(y, W1, byw)

    zacc, cnts = _sc_aggregate(yw, ewba, tailt, src, dst)

    BZ = 1000
    h = pl.pallas_call(
        _final_body,
        grid=(N // BZ,),
        in_specs=[
            pl.BlockSpec((NC, BZ, D_MSG), lambda i: (0, i, 0)),
            pl.BlockSpec((1, NC, NS, BZ), lambda i: (i, 0, 0, 0)),
            pl.BlockSpec((D_MSG, D_OUT), lambda i: (0, 0)),
            pl.BlockSpec((1, D_OUT), lambda i: (0, 0)),
        ],
        out_specs=pl.BlockSpec((BZ, D_OUT), lambda i: (i, 0)),
        out_shape=jax.ShapeDtypeStruct((N, D_OUT), jnp.float32),
    )(zacc, cnts.reshape(NC, NS, N // BZ, BZ).transpose(2, 0, 1, 3),
      W_upd, b_upd.reshape(1, D_OUT))

    return h


# speedup vs baseline: 4.2513x; 1.0264x over previous
import jax
import jax.numpy as jnp
from jax import lax
from jax.experimental import pallas as pl
from jax.experimental.pallas import tpu as pltpu
from jax.experimental.pallas import tpu_sc as plsc

N = 10000
E = 320000
D_NODE = 128
D_EDGE = 16
D_OUT = 128
D_MSG = 144

NC = 2
NS = 16
NW = NC * NS
EPW = E // NW
CHUNK = 40
NCHUNK = EPW // CHUNK
ZROWS = N // NS
ZTAIL = ZROWS % CHUNK
TTILES = E // 128


def _edge_body(eT_ref, wa_ref, ba_ref, v_ref, oa_ref, ot_ref):
    eT = eT_ref[...]
    oa_ref[...] = (
        lax.dot_general(eT, wa_ref[...], (((0,), (0,)), ((), ())),
                        preferred_element_type=jnp.float32)
        + ba_ref[...]
    )
    ft = lax.dot_general(v_ref[...], eT, (((0,), (0,)), ((), ())),
                         preferred_element_type=jnp.float32)
    for ct in range(ot_ref.shape[0]):
        ot_ref[ct] = ft[:, 128 * ct:128 * (ct + 1)]


def _yw_body(y_ref, w_ref, b_ref, o_ref):
    o_ref[...] = jnp.dot(y_ref[...], w_ref[...],
                         preferred_element_type=jnp.float32) + b_ref[...]


def _final_body(z_ref, c_ref, w_ref, b_ref, o_ref):
    zs = z_ref[0] + z_ref[1]
    cnt = jnp.maximum(jnp.sum(c_ref[0], axis=(0, 1)), 1.0)
    zm = zs / cnt[:, None]
    o_ref[...] = jnp.maximum(
        jnp.dot(zm, w_ref[...], preferred_element_type=jnp.float32)
        + b_ref[...],
        0.0,
    )


def _sc_body(yw_hbm, ewba_hbm, tail_hbm, src_hbm, dst_hbm, z_hbm, cnt_hbm,
             zsh, srcv0, srcv1, dstv0, dstv1, dsts0, dsts1, eav0, eav1,
             tlv0, tlv1, mbuf0, mbuf1, hist,
             sem_in0, sem_in1, sem_g0, sem_g1, sem_s0, sem_s1):
    c = lax.axis_index("c")
    s = lax.axis_index("s")
    wid = c * NS + s
    base = wid * EPW

    srcv = (srcv0, srcv1)
    dstv = (dstv0, dstv1)
    eav = (eav0, eav1)
    tlv = (tlv0, tlv1)
    mbuf = (mbuf0, mbuf1)
    dsts = (dsts0, dsts1)
    sem_in = (sem_in0, sem_in1)
    sem_g = (sem_g0, sem_g1)
    sem_s = (sem_s0, sem_s1)

    zero16 = jnp.zeros((16,), jnp.float32)
    one16 = jnp.full((16,), 1.0, jnp.float32)
    iota16 = lax.iota(jnp.int32, 16)

    def zero_row(r, carry):
        for j in range(D_MSG // 16):
            mbuf0[r, pl.ds(j * 16, 16)] = zero16
        return carry
    lax.fori_loop(0, CHUNK, zero_row, 0)

    def zero_hist(i, carry):
        hist[pl.ds(i * 16, 16)] = zero16
        return carry
    lax.fori_loop(0, N // 16, zero_hist, 0)

    for t in range(ZROWS // CHUNK):
        pltpu.sync_copy(mbuf0, zsh.at[pl.ds(s * ZROWS + t * CHUNK, CHUNK)])
    pltpu.sync_copy(mbuf0.at[pl.ds(0, ZTAIL)],
                    zsh.at[pl.ds(s * ZROWS + (ZROWS // CHUNK) * CHUNK,
                                 ZTAIL)])
    plsc.subcore_barrier()

    def chunk_off(g):
        off = base + g * CHUNK
        ct0 = jnp.minimum(lax.shift_right_logical(off, 7), TTILES - 2)
        return off, ct0

    def in_copies(g, b):
        off, ct0 = chunk_off(g)
        return (
            pltpu.make_async_copy(src_hbm.at[pl.ds(off, CHUNK)], srcv[b],
                                  sem_in[b]),
            pltpu.make_async_copy(dst_hbm.at[pl.ds(off, CHUNK)], dstv[b],
                                  sem_in[b]),
            pltpu.make_async_copy(ewba_hbm.at[pl.ds(off, CHUNK)], eav[b],
                                  sem_in[b]),
            pltpu.make_async_copy(tail_hbm.at[ct0], tlv[b].at[0],
                                  sem_in[b]),
        )

    def tail2_copy(g, b):
        off, ct0 = chunk_off(g)
        return pltpu.make_async_copy(tail_hbm.at[ct0 + 1], tlv[b].at[1],
                                     sem_in[b])

    def crosses(g):
        off, ct0 = chunk_off(g)
        return off - lax.shift_left(ct0, 7) + CHUNK - 1 >= 128

    def in_start(g, b):
        for cp in in_copies(g, b):
            cp.start()

        @pl.when(crosses(g))
        def _():
            tail2_copy(g, b).start()

    def in_wait(g, b):
        for cp in in_copies(g, b):
            cp.wait()

        @pl.when(crosses(g))
        def _():
            tail2_copy(g, b).wait()

    def gather(b):
        return pltpu.make_async_copy(yw_hbm.at[srcv[b]], mbuf[b], sem_g[b])

    def compute(g, b):
        off, ct0 = chunk_off(g)
        c_off = off - lax.shift_left(ct0, 7)

        @plsc.parallel_loop(0, CHUNK, unroll=2)
        def row(r):
            q = c_off + r
            t = lax.shift_right_logical(q, 7)
            l = q - lax.shift_left(t, 7)
            tail = plsc.load_gather(
                tlv[b],
                [jnp.full((16,), t, jnp.int32), iota16,
                 jnp.full((16,), l, jnp.int32)])
            ea = [eav[b][r, pl.ds(j * 16, 16)] for j in range(8)]
            yv = [mbuf[b][r, pl.ds(j * 16, 16)] for j in range(9)]
            res = [jnp.maximum(ea[j] + yv[j], 0.0) for j in range(8)]
            res.append(jnp.maximum(tail + yv[8], 0.0))
            for j in range(9):
                mbuf[b][r, pl.ds(j * 16, 16)] = res[j]

        plsc.addupdate_scatter(hist, [dstv[b][pl.ds(0, 16)]], one16)
        plsc.addupdate_scatter(hist, [dstv[b][pl.ds(16, 16)]], one16)
        plsc.addupdate_scatter(hist, [dstv[b][pl.ds(24, 16)]], one16,
                               mask=iota16 >= 8)

    in_start(0, 0)
    in_start(1, 1)
    in_wait(0, 0)
    gather(0).start()

    def scatter_wait(b):
        pltpu.make_async_copy(mbuf[b], zsh.at[dsts[b]], sem_s[b]).wait()

    def step(g, b):
        nb = 1 - b

        @pl.when(g + 1 < NCHUNK)
        def _():
            in_wait(g + 1, nb)

            @pl.when(g >= 1)
            def _():
                scatter_wait(nb)

            gather(nb).start()

        gather(b).wait()
        compute(g, b)
        dsts[b][pl.ds(0, 16)] = dstv[b][pl.ds(0, 16)]
        dsts[b][pl.ds(16, 16)] = dstv[b][pl.ds(16, 16)]
        dsts[b][pl.ds(24, 16)] = dstv[b][pl.ds(24, 16)]
        pltpu.async_copy(mbuf[b], zsh.at[dsts[b]], sem_s[b], add=True)

        @pl.when(g + 2 < NCHUNK)
        def _():
            in_start(g + 2, b)

    def pair_body(i, carry):
        step(2 * i, 0)
        step(2 * i + 1, 1)
        return carry

    lax.fori_loop(0, NCHUNK // 2, pair_body, 0)
    scatter_wait(0)
    scatter_wait(1)
    plsc.subcore_barrier()

    for t in range(ZROWS // CHUNK):
        r0 = s * ZROWS + t * CHUNK
        pltpu.sync_copy(zsh.at[pl.ds(r0, CHUNK)], mbuf0)
        pltpu.sync_copy(mbuf0, z_hbm.at[c, pl.ds(r0, CHUNK)])
    r0 = s * ZROWS + (ZROWS // CHUNK) * CHUNK
    pltpu.sync_copy(zsh.at[pl.ds(r0, ZTAIL)], mbuf0.at[pl.ds(0, ZTAIL)])
    pltpu.sync_copy(mbuf0.at[pl.ds(0, ZTAIL)], z_hbm.at[c, pl.ds(r0, ZTAIL)])
    pltpu.sync_copy(hist, cnt_hbm.at[c, s])


_sc_aggregate = pl.kernel(
    _sc_body,
    out_type=(
        jax.ShapeDtypeStruct((NC, N, D_MSG), jnp.float32),
        jax.ShapeDtypeStruct((NC, NS, N), jnp.float32),
    ),
    mesh=plsc.VectorSubcoreMesh(core_axis_name="c", subcore_axis_name="s"),
    compiler_params=pltpu.CompilerParams(use_tc_tiling_on_sc=False,
                                         needs_layout_passes=False),
    scratch_types=[
        pltpu.VMEM_SHARED((N, D_MSG), jnp.float32),
        pltpu.VMEM((CHUNK,), jnp.int32),
        pltpu.VMEM((CHUNK,), jnp.int32),
        pltpu.VMEM((CHUNK,), jnp.int32),
        pltpu.VMEM((CHUNK,), jnp.int32),
        pltpu.VMEM((CHUNK,), jnp.int32),
        pltpu.VMEM((CHUNK,), jnp.int32),
        pltpu.VMEM((CHUNK, D_NODE), jnp.float32),
        pltpu.VMEM((CHUNK, D_NODE), jnp.float32),
        pltpu.VMEM((2, 16, 128), jnp.float32),
        pltpu.VMEM((2, 16, 128), jnp.float32),
        pltpu.VMEM((CHUNK, D_MSG), jnp.float32),
        pltpu.VMEM((CHUNK, D_MSG), jnp.float32),
        pltpu.VMEM((N,), jnp.float32),
        pltpu.SemaphoreType.DMA,
        pltpu.SemaphoreType.DMA,
        pltpu.SemaphoreType.DMA,
        pltpu.SemaphoreType.DMA,
        pltpu.SemaphoreType.DMA,
        pltpu.SemaphoreType.DMA,
    ],
)


def kernel(y, edge_attr, edge_index, W_pre, b_pre, W_upd, b_upd):
    src = edge_index[0]
    dst = edge_index[1]
    eT = edge_attr.T
    W1 = W_pre[:D_NODE]
    byw = jnp.zeros((1, D_MSG), jnp.float32).at[0, D_NODE:].set(
        b_pre[D_NODE:])
    W2a = W_pre[D_NODE:, :D_NODE]
    ba = b_pre[:D_NODE].reshape(1, D_NODE)
    Vmsg = W_pre[D_NODE:, D_NODE:]

    BE = 6400
    BT = BE // 128
    ewba, tailt = pl.pallas_call(
        _edge_body,
        grid=(E // BE,),
        in_specs=[
            pl.BlockSpec((D_EDGE, BE), lambda i: (0, i)),
            pl.BlockSpec((D_EDGE, D_NODE), lambda i: (0, 0)),
            pl.BlockSpec((1, D_NODE), lambda i: (0, 0)),
            pl.BlockSpec((D_EDGE, D_EDGE), lambda i: (0, 0)),
        ],
        out_specs=[
            pl.BlockSpec((BE, D_NODE), lambda i: (i, 0)),
            pl.BlockSpec((BT, 16, 128), lambda i: (i, 0, 0)),
        ],
        out_shape=[
            jax.ShapeDtypeStruct((E, D_NODE), jnp.float32),
            jax.ShapeDtypeStruct((TTILES, 16, 128), jnp.float32),
        ],
    )(eT, W2a, ba, Vmsg)

    BN = 2000
    yw = pl.pallas_call(
        _yw_body,
        grid=(N // BN,),
        in_specs=[
            pl.BlockSpec((BN, D_NODE), lambda i: (i, 0)),
            pl.BlockSpec((D_NODE, D_MSG), lambda i: (0, 0)),
            pl.BlockSpec((1, D_MSG), lambda i: (0, 0)),
        ],
        out_specs=pl.BlockSpec((BN, D_MSG), lambda i: (i, 0)),
        out_shape=jax.ShapeDtypeStruct((N, D_MSG), jnp.float32),
    )(y, W1, byw)

    zacc, cnts = _sc_aggregate(yw, ewba, tailt, src, dst)

    BZ = 1000
    h = pl.pallas_call(
        _final_body,
        grid=(N // BZ,),
        in_specs=[
            pl.BlockSpec((NC, BZ, D_MSG), lambda i: (0, i, 0)),
            pl.BlockSpec((1, NC, NS, BZ), lambda i: (i, 0, 0, 0)),
            pl.BlockSpec((D_MSG, D_OUT), lambda i: (0, 0)),
            pl.BlockSpec((1, D_OUT), lambda i: (0, 0)),
        ],
        out_specs=pl.BlockSpec((BZ, D_OUT), lambda i: (i, 0)),
        out_shape=jax.ShapeDtypeStruct((N, D_OUT), jnp.float32),
    )(zacc, cnts.reshape(NC, NS, N // BZ, BZ).transpose(2, 0, 1, 3),
      W_upd, b_upd.reshape(1, D_OUT))

    return h
